# baseline (speedup 1.0000x reference)
import functools

import jax
import jax.numpy as jnp
from jax import lax
from jax.experimental import pallas as pl
from jax.experimental.pallas import tpu as pltpu
from jax.experimental.pallas import tpu_sc as plsc

N = 10000
E = 160000
EDGE_DIM = 16
NODE_DIM = 256
HIDDEN = 256

NC = 2
NS = 16
NW = NC * NS
CHUNK = 128
NCHUNKS = E // CHUNK
BASE_CHUNKS = NCHUNKS // NW
EXTRA = NCHUNKS - BASE_CHUNKS * NW
MAXC = BASE_CHUNKS + 1
NPAD = 10240
SLAB = NPAD // NS
CROWS = NPAD // EDGE_DIM
CSLAB = CROWS // NS


def _sc_body(edge_hbm, ei_hbm, zeros_hbm, rowidx_hbm,
             sums_hbm, cnt_hbm,
             edge_buf, idx_v, hist, idxrow, sum_tab, cnt_tab):
    cid = lax.axis_index("c")
    sid = lax.axis_index("s")
    wid = sid * NC + cid

    pltpu.sync_copy(zeros_hbm, hist)
    pltpu.sync_copy(rowidx_hbm, idxrow)
    pltpu.sync_copy(hist, sum_tab.at[pl.ds(sid * SLAB, SLAB)])
    pltpu.sync_copy(hist.at[pl.ds(0, CSLAB)], cnt_tab.at[pl.ds(sid * CSLAB, CSLAB)])
    plsc.subcore_barrier()

    base = wid * BASE_CHUNKS
    pltpu.sync_copy(edge_hbm.at[pl.ds(base * CHUNK, BASE_CHUNKS * CHUNK)],
                    edge_buf.at[pl.ds(0, BASE_CHUNKS * CHUNK)])
    pltpu.sync_copy(ei_hbm.at[1, pl.ds(base * CHUNK, BASE_CHUNKS * CHUNK)],
                    idx_v.at[pl.ds(0, BASE_CHUNKS * CHUNK)])

    @pl.when(wid < EXTRA)
    def _load_extra():
        xc = NW * BASE_CHUNKS + wid
        pltpu.sync_copy(edge_hbm.at[pl.ds(xc * CHUNK, CHUNK)],
                        edge_buf.at[pl.ds(BASE_CHUNKS * CHUNK, CHUNK)])
        pltpu.sync_copy(ei_hbm.at[1, pl.ds(xc * CHUNK, CHUNK)],
                        idx_v.at[pl.ds(BASE_CHUNKS * CHUNK, CHUNK)])

    ones16 = jnp.ones((16,), jnp.float32)

    def _do_chunk(j):
        pltpu.sync_copy(edge_buf.at[pl.ds(j * CHUNK, CHUNK)],
                        sum_tab.at[idx_v.at[pl.ds(j * CHUNK, CHUNK)]],
                        add=True)
        def _cnt(t, _):
            v = idx_v[pl.ds(j * CHUNK + t * 16, 16)]
            plsc.addupdate_scatter(
                hist, [lax.shift_right_logical(v, 4), lax.bitwise_and(v, 15)],
                ones16)
            return 0
        lax.fori_loop(0, CHUNK // 16, _cnt, 0)

    lax.fori_loop(0, BASE_CHUNKS, lambda j, _: (_do_chunk(j), 0)[1], 0)

    @pl.when(wid < EXTRA)
    def _extra_chunk():
        _do_chunk(BASE_CHUNKS)

    def _merge(k, _):
        pltpu.sync_copy(hist.at[pl.ds(k * CHUNK, CHUNK)],
                        cnt_tab.at[idxrow.at[pl.ds(k * CHUNK, CHUNK)]],
                        add=True)
        return 0
    lax.fori_loop(0, CROWS // CHUNK, _merge, 0)
    plsc.subcore_barrier()

    pltpu.sync_copy(sum_tab.at[pl.ds(sid * SLAB, SLAB)],
                    sums_hbm.at[cid, pl.ds(sid * SLAB, SLAB)])
    pltpu.sync_copy(cnt_tab.at[pl.ds(sid * CSLAB, CSLAB)],
                    cnt_hbm.at[cid, pl.ds(sid * CSLAB, CSLAB)])


_sc_scatter = functools.partial(
    pl.kernel,
    out_type=(jax.ShapeDtypeStruct((NC, NPAD, EDGE_DIM), jnp.float32),
              jax.ShapeDtypeStruct((NC, CROWS, EDGE_DIM), jnp.float32)),
    mesh=plsc.VectorSubcoreMesh(core_axis_name="c", subcore_axis_name="s",
                                num_cores=NC, num_subcores=NS),
    scratch_types=[
        pltpu.VMEM((MAXC * CHUNK, EDGE_DIM), jnp.float32),
        pltpu.VMEM((MAXC * CHUNK,), jnp.int32),
        pltpu.VMEM((CROWS, EDGE_DIM), jnp.float32),
        pltpu.VMEM((CROWS,), jnp.int32),
        pltpu.VMEM_SHARED((NPAD, EDGE_DIM), jnp.float32),
        pltpu.VMEM_SHARED((CROWS, EDGE_DIM), jnp.float32),
    ],
    compiler_params=pltpu.CompilerParams(use_tc_tiling_on_sc=False,
                                         needs_layout_passes=False),
)(_sc_body)


ROWS_BLK = 400
GRID = N // ROWS_BLK


def _mm_body(x_ref, w_ref, o_ref):
    o_ref[...] = jnp.dot(x_ref[...], w_ref[...],
                         preferred_element_type=jnp.float32)


def _tc_mm(x, w1x):
    return pl.pallas_call(
        _mm_body,
        grid=(GRID,),
        in_specs=[
            pl.BlockSpec((ROWS_BLK, NODE_DIM), lambda i: (i, 0)),
            pl.BlockSpec((NODE_DIM, HIDDEN), lambda i: (0, 0)),
        ],
        out_specs=pl.BlockSpec((ROWS_BLK, HIDDEN), lambda i: (i, 0)),
        out_shape=jax.ShapeDtypeStruct((N, HIDDEN), jnp.float32),
        compiler_params=pltpu.CompilerParams(
            dimension_semantics=("arbitrary",)),
    )(x, w1x)


def _mlp_body(h0_ref, s_ref, c_ref, w1a_ref, b1_ref, g_ref, be_ref,
              w2_ref, b2_ref, o_ref):
    cnt = jnp.maximum(c_ref[0] + c_ref[1], 1.0)
    agg = (s_ref[0] + s_ref[1]) / cnt
    h = h0_ref[...]
    h = h + jnp.dot(agg, w1a_ref[...], preferred_element_type=jnp.float32)
    h = h + b1_ref[...]
    h = jnp.maximum(h, 0.0)
    mu = jnp.mean(h, axis=1, keepdims=True)
    d = h - mu
    var = jnp.mean(d * d, axis=1, keepdims=True)
    hn = d * lax.rsqrt(var + 1e-5) * g_ref[...] + be_ref[...]
    o_ref[...] = (jnp.dot(hn, w2_ref[...], preferred_element_type=jnp.float32)
                  + b2_ref[...])


def _tc_mlp(h0, sums, cnt3, w1a, b1, gamma, beta, w2, b2):
    full = lambda shape: pl.BlockSpec(shape, lambda i: (0,) * len(shape))
    return pl.pallas_call(
        _mlp_body,
        grid=(GRID,),
        in_specs=[
            pl.BlockSpec((ROWS_BLK, HIDDEN), lambda i: (i, 0)),
            pl.BlockSpec((NC, ROWS_BLK, EDGE_DIM), lambda i: (0, i, 0)),
            pl.BlockSpec((NC, ROWS_BLK, 1), lambda i: (0, i, 0)),
            full((EDGE_DIM, HIDDEN)),
            full((1, HIDDEN)),
            full((1, HIDDEN)),
            full((1, HIDDEN)),
            full((HIDDEN, HIDDEN)),
            full((1, HIDDEN)),
        ],
        out_specs=pl.BlockSpec((ROWS_BLK, HIDDEN), lambda i: (i, 0)),
        out_shape=jax.ShapeDtypeStruct((N, HIDDEN), jnp.float32),
        compiler_params=pltpu.CompilerParams(
            dimension_semantics=("arbitrary",)),
    )(h0, sums, cnt3, w1a, b1, gamma, beta, w2, b2)


def kernel(x, edge_index, edge_attr, W1, b1, gamma, beta, W2, b2):
    ei = edge_index.astype(jnp.int32)
    zeros_c = jnp.zeros((CROWS, EDGE_DIM), jnp.float32)
    rowidx_c = jnp.arange(CROWS, dtype=jnp.int32)
    sums, cnt = _sc_scatter(edge_attr, ei, zeros_c, rowidx_c)
    h0 = _tc_mm(x, W1[:NODE_DIM])
    cnt3 = cnt.reshape(NC, NPAD, 1)
    return _tc_mlp(h0, sums, cnt3, W1[NODE_DIM:],
                   b1.reshape(1, HIDDEN), gamma.reshape(1, HIDDEN),
                   beta.reshape(1, HIDDEN), W2, b2.reshape(1, HIDDEN))


# speedup vs baseline: 1.3284x; 1.3284x over previous
import functools

import jax
import jax.numpy as jnp
from jax import lax
from jax.experimental import pallas as pl
from jax.experimental.pallas import tpu as pltpu
from jax.experimental.pallas import tpu_sc as plsc

N = 10000
E = 160000
EDGE_DIM = 16
NODE_DIM = 256
HIDDEN = 256

NC = 2
NS = 16
NW = NC * NS
CHUNK = 128
NCHUNKS = E // CHUNK
BASE_CHUNKS = NCHUNKS // NW
EXTRA = NCHUNKS - BASE_CHUNKS * NW
MAXC = BASE_CHUNKS + 1
BATCH0 = 20
NPAD = 10240
SLAB = NPAD // NS
CROWS = NPAD // EDGE_DIM
CSLAB = CROWS // NS


def _sc_body(ea4_hbm, ei4_hbm, zeros_hbm, rowidx_hbm,
             sums_hbm, cnt_hbm,
             tile_buf, edge_buf, idx2d, hist, idxrow, sum_tab, cnt_tab):
    cid = lax.axis_index("c")
    sid = lax.axis_index("s")
    wid = sid * NC + cid

    pltpu.sync_copy(zeros_hbm, hist)
    pltpu.sync_copy(rowidx_hbm, idxrow)
    pltpu.sync_copy(hist, sum_tab.at[pl.ds(sid * SLAB, SLAB)])
    pltpu.sync_copy(hist.at[pl.ds(0, CSLAB)], cnt_tab.at[pl.ds(sid * CSLAB, CSLAB)])
    plsc.subcore_barrier()

    base = wid * BASE_CHUNKS
    pltpu.sync_copy(ei4_hbm.at[pl.ds(base, BASE_CHUNKS), 1],
                    idx2d.at[pl.ds(0, BASE_CHUNKS)])

    @pl.when(wid < EXTRA)
    def _load_extra_idx():
        pltpu.sync_copy(ei4_hbm.at[pl.ds(NW * BASE_CHUNKS + wid, 1), 1],
                        idx2d.at[pl.ds(BASE_CHUNKS, 1)])

    iota16 = lax.iota(jnp.int32, 16)
    ones16 = jnp.ones((16,), jnp.float32)

    def _stage(tr, c_lo, nb, buf_lo):
        pltpu.sync_copy(ea4_hbm.at[tr, pl.ds(c_lo, nb)],
                        tile_buf.at[pl.ds(0, nb)])

        def _tc_loop(tc, _):
            e_base = (buf_lo + tc) * CHUNK + iota16

            def _r_loop(r, _):
                f_idx = jnp.full((16,), tr * 8, jnp.int32) + r

                def _c_loop(c0, _):
                    x = tile_buf[tc, r, pl.ds(c0 * 16, 16)]
                    plsc.store_scatter(edge_buf, [e_base + c0 * 16, f_idx], x)
                    return 0
                lax.fori_loop(0, 8, _c_loop, 0)
                return 0
            lax.fori_loop(0, 8, _r_loop, 0)
            return 0
        lax.fori_loop(0, nb, _tc_loop, 0)

    for b_lo, b_n in ((0, BATCH0), (BATCH0, BASE_CHUNKS - BATCH0)):
        for tr in (0, 1):
            _stage(tr, base + b_lo, b_n, b_lo)

    @pl.when(wid < EXTRA)
    def _stage_extra():
        for tr in (0, 1):
            _stage(tr, NW * BASE_CHUNKS + wid, 1, BASE_CHUNKS)

    def _do_chunk(j):
        pltpu.sync_copy(edge_buf.at[pl.ds(j * CHUNK, CHUNK)],
                        sum_tab.at[idx2d.at[j]], add=True)

        def _cnt(t, _):
            v = idx2d[j, pl.ds(t * 16, 16)]
            plsc.addupdate_scatter(
                hist, [lax.shift_right_logical(v, 4), lax.bitwise_and(v, 15)],
                ones16)
            return 0
        lax.fori_loop(0, CHUNK // 16, _cnt, 0)

    lax.fori_loop(0, BASE_CHUNKS, lambda j, _: (_do_chunk(j), 0)[1], 0)

    @pl.when(wid < EXTRA)
    def _extra_chunk():
        _do_chunk(BASE_CHUNKS)

    def _merge(k, _):
        pltpu.sync_copy(hist.at[pl.ds(k * CHUNK, CHUNK)],
                        cnt_tab.at[idxrow.at[pl.ds(k * CHUNK, CHUNK)]],
                        add=True)
        return 0
    lax.fori_loop(0, CROWS // CHUNK, _merge, 0)
    plsc.subcore_barrier()

    pltpu.sync_copy(sum_tab.at[pl.ds(sid * SLAB, SLAB)],
                    sums_hbm.at[cid, pl.ds(sid * SLAB, SLAB)])
    pltpu.sync_copy(cnt_tab.at[pl.ds(sid * CSLAB, CSLAB)],
                    cnt_hbm.at[cid, pl.ds(sid * CSLAB, CSLAB)])


_sc_scatter = functools.partial(
    pl.kernel,
    out_type=(jax.ShapeDtypeStruct((NC, NPAD, EDGE_DIM), jnp.float32),
              jax.ShapeDtypeStruct((NC, CROWS, EDGE_DIM), jnp.float32)),
    mesh=plsc.VectorSubcoreMesh(core_axis_name="c", subcore_axis_name="s",
                                num_cores=NC, num_subcores=NS),
    scratch_types=[
        pltpu.VMEM((BATCH0, 8, CHUNK), jnp.float32),
        pltpu.VMEM((MAXC * CHUNK, EDGE_DIM), jnp.float32),
        pltpu.VMEM((MAXC, CHUNK), jnp.int32),
        pltpu.VMEM((CROWS, EDGE_DIM), jnp.float32),
        pltpu.VMEM((CROWS,), jnp.int32),
        pltpu.VMEM_SHARED((NPAD, EDGE_DIM), jnp.float32),
        pltpu.VMEM_SHARED((CROWS, EDGE_DIM), jnp.float32),
    ],
    compiler_params=pltpu.CompilerParams(use_tc_tiling_on_sc=False,
                                         needs_layout_passes=False),
)(_sc_body)


ROWS_BLK = 400
GRID = N // ROWS_BLK


def _mm_body(x_ref, w_ref, o_ref):
    o_ref[...] = jnp.dot(x_ref[...], w_ref[...],
                         preferred_element_type=jnp.float32)


def _tc_mm(x, w1x):
    return pl.pallas_call(
        _mm_body,
        grid=(GRID,),
        in_specs=[
            pl.BlockSpec((ROWS_BLK, NODE_DIM), lambda i: (i, 0)),
            pl.BlockSpec((NODE_DIM, HIDDEN), lambda i: (0, 0)),
        ],
        out_specs=pl.BlockSpec((ROWS_BLK, HIDDEN), lambda i: (i, 0)),
        out_shape=jax.ShapeDtypeStruct((N, HIDDEN), jnp.float32),
        compiler_params=pltpu.CompilerParams(
            dimension_semantics=("arbitrary",)),
    )(x, w1x)


def _mlp_body(h0_ref, s_ref, c_ref, w1a_ref, b1_ref, g_ref, be_ref,
              w2_ref, b2_ref, o_ref):
    cnt = jnp.maximum(c_ref[0] + c_ref[1], 1.0)
    agg = (s_ref[0] + s_ref[1]) / cnt
    h = h0_ref[...]
    h = h + jnp.dot(agg, w1a_ref[...], preferred_element_type=jnp.float32)
    h = h + b1_ref[...]
    h = jnp.maximum(h, 0.0)
    mu = jnp.mean(h, axis=1, keepdims=True)
    d = h - mu
    var = jnp.mean(d * d, axis=1, keepdims=True)
    hn = d * lax.rsqrt(var + 1e-5) * g_ref[...] + be_ref[...]
    o_ref[...] = (jnp.dot(hn, w2_ref[...], preferred_element_type=jnp.float32)
                  + b2_ref[...])


def _tc_mlp(h0, sums, cnt3, w1a, b1, gamma, beta, w2, b2):
    full = lambda shape: pl.BlockSpec(shape, lambda i: (0,) * len(shape))
    return pl.pallas_call(
        _mlp_body,
        grid=(GRID,),
        in_specs=[
            pl.BlockSpec((ROWS_BLK, HIDDEN), lambda i: (i, 0)),
            pl.BlockSpec((NC, ROWS_BLK, EDGE_DIM), lambda i: (0, i, 0)),
            pl.BlockSpec((NC, ROWS_BLK, 1), lambda i: (0, i, 0)),
            full((EDGE_DIM, HIDDEN)),
            full((1, HIDDEN)),
            full((1, HIDDEN)),
            full((1, HIDDEN)),
            full((HIDDEN, HIDDEN)),
            full((1, HIDDEN)),
        ],
        out_specs=pl.BlockSpec((ROWS_BLK, HIDDEN), lambda i: (i, 0)),
        out_shape=jax.ShapeDtypeStruct((N, HIDDEN), jnp.float32),
        compiler_params=pltpu.CompilerParams(
            dimension_semantics=("arbitrary",)),
    )(h0, sums, cnt3, w1a, b1, gamma, beta, w2, b2)


def kernel(x, edge_index, edge_attr, W1, b1, gamma, beta, W2, b2):
    ea4 = edge_attr.reshape(NCHUNKS, CHUNK, 2, 8).transpose(2, 0, 3, 1)
    ei4 = edge_index.astype(jnp.int32).reshape(2, NCHUNKS, CHUNK).transpose(1, 0, 2)
    zeros_c = jnp.zeros((CROWS, EDGE_DIM), jnp.float32)
    rowidx_c = jnp.arange(CROWS, dtype=jnp.int32)
    sums, cnt = _sc_scatter(ea4, ei4, zeros_c, rowidx_c)
    h0 = _tc_mm(x, W1[:NODE_DIM])
    cnt3 = cnt.reshape(NC, NPAD, 1)
    return _tc_mlp(h0, sums, cnt3, W1[NODE_DIM:],
                   b1.reshape(1, HIDDEN), gamma.reshape(1, HIDDEN),
                   beta.reshape(1, HIDDEN), W2, b2.reshape(1, HIDDEN))


# speedup vs baseline: 1.4562x; 1.0962x over previous
import functools

import jax
import jax.numpy as jnp
from jax import lax
from jax.experimental import pallas as pl
from jax.experimental.pallas import tpu as pltpu
from jax.experimental.pallas import tpu_sc as plsc

N = 10000
E = 160000
EDGE_DIM = 16
NODE_DIM = 256
HIDDEN = 256

NC = 2
NS = 16
NW = NC * NS
CHUNK = 128
NCHUNKS = E // CHUNK
BASE_CHUNKS = NCHUNKS // NW
EXTRA = NCHUNKS - BASE_CHUNKS * NW
MAXC = BASE_CHUNKS + 1
BATCH0 = 20
FIRE = 13
NPAD = 10240
SLAB = NPAD // NS
CROWS = NPAD // EDGE_DIM
CSLAB = CROWS // NS


def _sc_body(ea4_hbm, ei4_hbm, zeros_hbm, rowidx_hbm,
             sums_hbm, cnt_hbm,
             tile_buf, edge_buf, idx2d, hist, idxrow, cnt2, dsem,
             sum_tab, cnt_tab):
    cid = lax.axis_index("c")
    sid = lax.axis_index("s")
    wid = sid * NC + cid

    pltpu.sync_copy(zeros_hbm, hist)
    pltpu.sync_copy(rowidx_hbm, idxrow)
    pltpu.sync_copy(hist, sum_tab.at[pl.ds(sid * SLAB, SLAB)])
    pltpu.sync_copy(hist.at[pl.ds(0, CSLAB)], cnt_tab.at[pl.ds(sid * CSLAB, CSLAB)])
    plsc.subcore_barrier()

    base = wid * BASE_CHUNKS
    pltpu.sync_copy(ei4_hbm.at[pl.ds(base, BASE_CHUNKS), 1],
                    idx2d.at[pl.ds(0, BASE_CHUNKS)])

    @pl.when(wid < EXTRA)
    def _load_extra_idx():
        pltpu.sync_copy(ei4_hbm.at[pl.ds(NW * BASE_CHUNKS + wid, 1), 1],
                        idx2d.at[pl.ds(BASE_CHUNKS, 1)])

    iota16 = lax.iota(jnp.int32, 16)
    ones16 = jnp.ones((16,), jnp.float32)
    f_rows = [jnp.full((16,), f, jnp.int32) for f in range(16)]
    g_rows = [jnp.full((16,), g, jnp.int32) for g in range(16)]

    def _stage(tr, c_lo, nb, buf_lo):
        pltpu.sync_copy(ea4_hbm.at[tr, pl.ds(c_lo, nb)],
                        tile_buf.at[pl.ds(0, nb)])

        def _tc_loop(tc, _):
            e_base = (buf_lo + tc) * CHUNK + iota16
            for r in range(8):
                for c0 in range(8):
                    x = tile_buf[tc, r, pl.ds(c0 * 16, 16)]
                    plsc.store_scatter(edge_buf,
                                       [e_base + c0 * 16, f_rows[tr * 8 + r]],
                                       x)
            return 0
        lax.fori_loop(0, nb, _tc_loop, 0)

    def _counts(j):
        for t in range(CHUNK // 16):
            v = idx2d[j, pl.ds(t * 16, 16)]
            plsc.addupdate_scatter(
                hist, [lax.shift_right_logical(v, 4), lax.bitwise_and(v, 15)],
                ones16)

    for c_lo, nb in ((0, BATCH0), (BATCH0, BASE_CHUNKS - BATCH0)):
        for tr in (0, 1):
            _stage(tr, base + c_lo, nb, 0)
        if nb < BATCH0:
            @pl.when(wid < EXTRA)
            def _stage_extra():
                for tr in (0, 1):
                    _stage(tr, NW * BASE_CHUNKS + wid, 1, nb)
        for g_lo in range(0, nb, FIRE):
            g_n = min(FIRE, nb - g_lo)
            cps = []
            for j in range(g_lo, g_lo + g_n):
                cps.append(pltpu.async_copy(
                    edge_buf.at[pl.ds(j * CHUNK, CHUNK)],
                    sum_tab.at[idx2d.at[c_lo + j]], dsem, add=True))
                _counts(c_lo + j)
            for cp in cps:
                cp.wait()

    @pl.when(wid < EXTRA)
    def _extra_chunk():
        pltpu.sync_copy(
            edge_buf.at[pl.ds((BASE_CHUNKS - BATCH0) * CHUNK, CHUNK)],
            sum_tab.at[idx2d.at[BASE_CHUNKS]], add=True)
        _counts(BASE_CHUNKS)

    for k in range(CROWS // CHUNK):
        pltpu.sync_copy(hist.at[pl.ds(k * CHUNK, CHUNK)],
                        cnt_tab.at[idxrow.at[pl.ds(k * CHUNK, CHUNK)]],
                        add=True)
    plsc.subcore_barrier()

    pltpu.sync_copy(cnt_tab.at[pl.ds(sid * CSLAB, CSLAB)],
                    hist.at[pl.ds(0, CSLAB)])

    def _expand(k, _):
        c16 = hist[k, pl.ds(0, 16)]
        for g in range(16):
            cnt2[k * 16 + g, pl.ds(0, 16)] = c16[g_rows[g]]
        return 0
    lax.fori_loop(0, CSLAB, _expand, 0)

    pltpu.sync_copy(sum_tab.at[pl.ds(sid * SLAB, SLAB)],
                    sums_hbm.at[cid, pl.ds(sid * SLAB, SLAB)])
    pltpu.sync_copy(cnt2, cnt_hbm.at[cid, pl.ds(sid * SLAB, SLAB)])


_sc_scatter = functools.partial(
    pl.kernel,
    out_type=(jax.ShapeDtypeStruct((NC, NPAD, EDGE_DIM), jnp.float32),
              jax.ShapeDtypeStruct((NC, NPAD, EDGE_DIM), jnp.float32)),
    mesh=plsc.VectorSubcoreMesh(core_axis_name="c", subcore_axis_name="s",
                                num_cores=NC, num_subcores=NS),
    scratch_types=[
        pltpu.VMEM((BATCH0, 8, CHUNK), jnp.float32),
        pltpu.VMEM((BATCH0 * CHUNK, EDGE_DIM), jnp.float32),
        pltpu.VMEM((MAXC, CHUNK), jnp.int32),
        pltpu.VMEM((CROWS, EDGE_DIM), jnp.float32),
        pltpu.VMEM((CROWS,), jnp.int32),
        pltpu.VMEM((SLAB, EDGE_DIM), jnp.float32),
        pltpu.SemaphoreType.DMA,
        pltpu.VMEM_SHARED((NPAD, EDGE_DIM), jnp.float32),
        pltpu.VMEM_SHARED((CROWS, EDGE_DIM), jnp.float32),
    ],
    compiler_params=pltpu.CompilerParams(use_tc_tiling_on_sc=False,
                                         needs_layout_passes=False),
)(_sc_body)


MM_BLK = 400


def _mm_body(x_ref, w_ref, o_ref):
    o_ref[...] = jnp.dot(x_ref[...], w_ref[...],
                         preferred_element_type=jnp.float32)


def _tc_mm(x, w1x):
    return pl.pallas_call(
        _mm_body,
        grid=(N // MM_BLK,),
        in_specs=[
            pl.BlockSpec((MM_BLK, NODE_DIM), lambda i: (i, 0)),
            pl.BlockSpec((NODE_DIM, HIDDEN), lambda i: (0, 0)),
        ],
        out_specs=pl.BlockSpec((MM_BLK, HIDDEN), lambda i: (i, 0)),
        out_shape=jax.ShapeDtypeStruct((N, HIDDEN), jnp.float32),
        compiler_params=pltpu.CompilerParams(
            dimension_semantics=("arbitrary",)),
    )(x, w1x)


ROWS_BLK = 640
GRID = (N + ROWS_BLK - 1) // ROWS_BLK


def _mlp_body(h0_ref, s_hbm, c_hbm, w1a_ref, b1_ref, g_ref, be_ref,
              w2_ref, b2_ref, o_ref, s_buf, c_buf, ssem, csem):
    i = pl.program_id(0)
    slot = lax.rem(i, 2)
    nslot = lax.rem(i + 1, 2)

    def s_copy(blk, sl):
        return pltpu.make_async_copy(
            s_hbm.at[:, pl.ds(blk * ROWS_BLK, ROWS_BLK)], s_buf.at[sl], ssem)

    def c_copy(blk, sl):
        return pltpu.make_async_copy(
            c_hbm.at[:, pl.ds(blk * ROWS_BLK, ROWS_BLK)], c_buf.at[sl], csem)

    @pl.when(i == 0)
    def _prologue():
        s_copy(0, 0).start()
        c_copy(0, 0).start()

    @pl.when(i + 1 < GRID)
    def _prefetch():
        s_copy(i + 1, nslot).start()
        c_copy(i + 1, nslot).start()

    s_copy(i, slot).wait()
    c_copy(i, slot).wait()

    s6 = s_buf[slot, 0] + s_buf[slot, 1]
    c6 = jnp.maximum(c_buf[slot, 0] + c_buf[slot, 1], 1.0)
    agg = s6 / c6
    h = h0_ref[...]
    h = h + jnp.dot(agg, w1a_ref[...], preferred_element_type=jnp.float32)
    h = h + b1_ref[...]
    h = jnp.maximum(h, 0.0)
    mu = jnp.mean(h, axis=1, keepdims=True)
    d = h - mu
    var = jnp.mean(d * d, axis=1, keepdims=True)
    hn = d * lax.rsqrt(var + 1e-5) * g_ref[...] + be_ref[...]
    o_ref[...] = (jnp.dot(hn, w2_ref[...], preferred_element_type=jnp.float32)
                  + b2_ref[...])


def _tc_mlp(h0, sums, cnt2, w1a, b1, gamma, beta, w2, b2):
    full = lambda shape: pl.BlockSpec(shape, lambda i: (0,) * len(shape))
    return pl.pallas_call(
        _mlp_body,
        grid=(GRID,),
        in_specs=[
            pl.BlockSpec((ROWS_BLK, HIDDEN), lambda i: (i, 0)),
            pl.BlockSpec(memory_space=pl.ANY),
            pl.BlockSpec(memory_space=pl.ANY),
            full((EDGE_DIM, HIDDEN)),
            full((1, HIDDEN)),
            full((1, HIDDEN)),
            full((1, HIDDEN)),
            full((HIDDEN, HIDDEN)),
            full((1, HIDDEN)),
        ],
        out_specs=pl.BlockSpec((ROWS_BLK, HIDDEN), lambda i: (i, 0)),
        out_shape=jax.ShapeDtypeStruct((N, HIDDEN), jnp.float32),
        scratch_shapes=[
            pltpu.VMEM((2, NC, ROWS_BLK, EDGE_DIM), jnp.float32),
            pltpu.VMEM((2, NC, ROWS_BLK, EDGE_DIM), jnp.float32),
            pltpu.SemaphoreType.DMA,
            pltpu.SemaphoreType.DMA,
        ],
        compiler_params=pltpu.CompilerParams(
            dimension_semantics=("arbitrary",)),
    )(h0, sums, cnt2, w1a, b1, gamma, beta, w2, b2)


def kernel(x, edge_index, edge_attr, W1, b1, gamma, beta, W2, b2):
    ea4 = edge_attr.reshape(NCHUNKS, CHUNK, 2, 8).transpose(2, 0, 3, 1)
    ei4 = edge_index.astype(jnp.int32).reshape(2, NCHUNKS, CHUNK).transpose(1, 0, 2)
    zeros_c = jnp.zeros((CROWS, EDGE_DIM), jnp.float32)
    rowidx_c = jnp.arange(CROWS, dtype=jnp.int32)
    sums, cnt2 = _sc_scatter(ea4, ei4, zeros_c, rowidx_c)
    h0 = _tc_mm(x, W1[:NODE_DIM])
    return _tc_mlp(h0, sums, cnt2, W1[NODE_DIM:],
                   b1.reshape(1, HIDDEN), gamma.reshape(1, HIDDEN),
                   beta.reshape(1, HIDDEN), W2, b2.reshape(1, HIDDEN))
